# baseline (speedup 1.0000x reference)
import jax
import jax.numpy as jnp
from jax.experimental import pallas as pl
from jax.experimental.pallas import tpu as pltpu

B = 16384
E_DIM = 64
LAG = 256


def _tc_body(hi, ri, ti, e_hbm, r_hbm, h_hbm, rel_hbm, t_hbm, sem):
    def gather(tbl, idx, out):
        @pl.loop(0, B)
        def _(j):
            s = idx[j]
            pltpu.make_async_copy(
                tbl.at[pl.ds(s, 1)], out.at[pl.ds(j, 1)], sem
            ).start()

            @pl.when(j >= LAG)
            def _():
                pltpu.make_async_copy(
                    tbl.at[pl.ds(0, 1)], out.at[pl.ds(j - LAG, 1)], sem
                ).wait()

        @pl.loop(B - LAG, B)
        def _(j):
            pltpu.make_async_copy(
                tbl.at[pl.ds(0, 1)], out.at[pl.ds(j, 1)], sem
            ).wait()

    gather(e_hbm, hi, h_hbm)
    gather(r_hbm, ri, rel_hbm)
    gather(e_hbm, ti, t_hbm)


@jax.jit
def kernel(sample_batch, E_emb, R_emb):
    idx = sample_batch.T
    h_idx, r_idx, t_idx = idx[0], idx[1], idx[2]

    out = jax.ShapeDtypeStruct((B, E_DIM), jnp.float32)
    grid_spec = pltpu.PrefetchScalarGridSpec(
        num_scalar_prefetch=3,
        grid=(1,),
        in_specs=[
            pl.BlockSpec(memory_space=pltpu.HBM),
            pl.BlockSpec(memory_space=pltpu.HBM),
        ],
        out_specs=[
            pl.BlockSpec(memory_space=pltpu.HBM),
            pl.BlockSpec(memory_space=pltpu.HBM),
            pl.BlockSpec(memory_space=pltpu.HBM),
        ],
        scratch_shapes=[pltpu.SemaphoreType.DMA],
    )
    head, relation, tail = pl.pallas_call(
        _tc_body,
        grid_spec=grid_spec,
        out_shape=(out, out, out),
    )(h_idx, r_idx, t_idx, E_emb, R_emb)
    return (head[:, None, :], relation[:, None, :], tail[:, None, :])
